# baseline (speedup 1.0000x reference)
import functools

import jax
import jax.numpy as jnp
from jax import lax
from jax.experimental import pallas as pl
from jax.experimental.pallas import tpu as pltpu
from jax.experimental.pallas import tpu_sc as plsc

_CH = 125


@functools.lru_cache(maxsize=None)
def _make_deg(Nn, E):
    per_tile = E // 32
    nch = per_tile // _CH
    mesh = plsc.VectorSubcoreMesh(core_axis_name="c", subcore_axis_name="s")

    @functools.partial(
        pl.kernel,
        out_type=jax.ShapeDtypeStruct((2, Nn), jnp.float32),
        mesh=mesh,
        scratch_types=[
            pltpu.VMEM((nch, _CH), jnp.int32),
            pltpu.VMEM((128,), jnp.float32),
            pltpu.VMEM((Nn,), jnp.float32),
            pltpu.VMEM_SHARED((Nn,), jnp.float32),
        ],
    )
    def degk(dst_hbm, out_hbm, dst_v, ones_v, dvmem, dacc):
        c = lax.axis_index("c")
        s = lax.axis_index("s")
        b = c * 16 + s
        pltpu.sync_copy(dst_hbm.at[b], dst_v)
        ov = jnp.ones((16,), jnp.float32)
        for k in range(8):
            ones_v[pl.ds(k * 16, 16)] = ov

        @pl.when(s == 0)
        def _init():
            zv = jnp.zeros((16,), jnp.float32)

            def z(i, carry):
                dvmem[pl.ds(i * 16, 16)] = zv
                return carry

            lax.fori_loop(0, Nn // 16, z, 0)
            pltpu.sync_copy(dvmem, dacc)

        plsc.subcore_barrier()

        def step(j, carry):
            pltpu.sync_copy(ones_v.at[pl.ds(0, _CH)], dacc.at[dst_v.at[j]], add=True)
            return carry

        lax.fori_loop(0, nch, step, 0)
        plsc.subcore_barrier()

        @pl.when(s == 0)
        def _export():
            pltpu.sync_copy(dacc, dvmem)
            pltpu.sync_copy(dvmem, out_hbm.at[c])

    return degk


@functools.lru_cache(maxsize=None)
def _make_edge_sum(Nn, D, nch):
    BLK = 16
    nblk = nch // BLK
    rows_per_tile = Nn // 16
    nz = rows_per_tile // _CH
    mesh = plsc.VectorSubcoreMesh(core_axis_name="c", subcore_axis_name="s")

    @functools.partial(
        pl.kernel,
        out_type=jax.ShapeDtypeStruct((2, Nn, D), jnp.float32),
        mesh=mesh,
        scratch_types=[
            pltpu.VMEM((BLK, _CH), jnp.int32),
            pltpu.VMEM((BLK, _CH), jnp.int32),
            pltpu.VMEM((_CH, D), jnp.float32),
            pltpu.VMEM((_CH, D), jnp.float32),
            pltpu.VMEM_SHARED((Nn, D), jnp.float32),
            pltpu.SemaphoreType.DMA,
            pltpu.SemaphoreType.DMA,
        ],
    )
    def edge_sum(g_hbm, src_hbm, dst_hbm, out_hbm, src_v, dst_v, rows0, rows1,
                 acc, sem0, sem1):
        c = lax.axis_index("c")
        s = lax.axis_index("s")

        zv = jnp.zeros((16,), jnp.float32)

        def zrow(i, carry):
            for k in range(D // 16):
                rows0[i, pl.ds(k * 16, 16)] = zv
            return carry

        lax.fori_loop(0, _CH, zrow, 0)

        def zcp(t, carry):
            pltpu.sync_copy(rows0, acc.at[pl.ds(s * rows_per_tile + t * _CH, _CH)])
            return carry

        lax.fori_loop(0, nz, zcp, 0)
        plsc.subcore_barrier()

        def blk_body(bi, carry):
            pltpu.sync_copy(src_hbm.at[c, s, pl.ds(bi * BLK, BLK)], src_v)
            pltpu.sync_copy(dst_hbm.at[s, pl.ds(bi * BLK, BLK)], dst_v)
            pltpu.async_copy(g_hbm.at[src_v.at[0]], rows0, sem0)
            pltpu.async_copy(g_hbm.at[src_v.at[1]], rows1, sem1)

            def step(i, carry2):
                j = 2 * i
                pltpu.make_async_copy(g_hbm.at[src_v.at[j]], rows0, sem0).wait()
                pltpu.sync_copy(rows0, acc.at[dst_v.at[j]], add=True)

                @pl.when(j + 2 < BLK)
                def _():
                    pltpu.async_copy(g_hbm.at[src_v.at[j + 2]], rows0, sem0)

                pltpu.make_async_copy(g_hbm.at[src_v.at[j + 1]], rows1, sem1).wait()
                pltpu.sync_copy(rows1, acc.at[dst_v.at[j + 1]], add=True)

                @pl.when(j + 3 < BLK)
                def _():
                    pltpu.async_copy(g_hbm.at[src_v.at[j + 3]], rows1, sem1)

                return carry2

            lax.fori_loop(0, BLK // 2, step, 0)
            return carry

        lax.fori_loop(0, nblk, blk_body, 0)
        plsc.subcore_barrier()

        base = s * 624

        def exp_chunk(off, sz):
            pltpu.sync_copy(acc.at[pl.ds(base + off, sz)],
                            rows0.at[pl.ds(0, sz)])
            pltpu.sync_copy(rows0.at[pl.ds(0, sz)],
                            out_hbm.at[c, pl.ds(base + off, sz)])

        for off in (0, 120, 240, 360, 480):
            exp_chunk(off, 120)

        @pl.when(s < 15)
        def _tail24():
            exp_chunk(600, 24)

        @pl.when(s == 15)
        def _tail40():
            exp_chunk(600, 40)

    return edge_sum


def _tc_pre(X2, degp3, W1):
    B, Nn, D = X2.shape
    BN = 1000
    NB = Nn // BN

    def body(x_ref, dp_ref, w_ref, o_ref):
        dp = dp_ref[...]
        dinv = lax.rsqrt(dp[0] + dp[1] + 1.0)
        h = jnp.dot(x_ref[0], w_ref[...], preferred_element_type=jnp.float32)
        o_ref[0] = h * dinv

    return pl.pallas_call(
        body,
        grid=(B, NB),
        in_specs=[
            pl.BlockSpec((1, BN, D), lambda c, i: (c, i, 0)),
            pl.BlockSpec((2, BN, 1), lambda c, i: (0, i, 0)),
            pl.BlockSpec((D, D), lambda c, i: (0, 0)),
        ],
        out_specs=pl.BlockSpec((1, BN, D), lambda c, i: (c, i, 0)),
        out_shape=jax.ShapeDtypeStruct((B, Nn, D), jnp.float32),
    )(X2, degp3, W1)


def _tc_mid(S1, g1, degp3, W2, b1r):
    B, Nn, D = S1.shape
    BN = 1000
    NB = Nn // BN

    def body(s_ref, g_ref, dp_ref, w_ref, b_ref, o_ref):
        dp = dp_ref[...]
        dinv = lax.rsqrt(dp[0] + dp[1] + 1.0)
        t = jnp.maximum((s_ref[0] + g_ref[0]) * dinv + b_ref[...], 0.0)
        h = jnp.dot(t, w_ref[...], preferred_element_type=jnp.float32)
        o_ref[0] = h * dinv

    return pl.pallas_call(
        body,
        grid=(B, NB),
        in_specs=[
            pl.BlockSpec((1, BN, D), lambda c, i: (c, i, 0)),
            pl.BlockSpec((1, BN, D), lambda c, i: (c, i, 0)),
            pl.BlockSpec((2, BN, 1), lambda c, i: (0, i, 0)),
            pl.BlockSpec((D, D), lambda c, i: (0, 0)),
            pl.BlockSpec((1, D), lambda c, i: (0, 0)),
        ],
        out_specs=pl.BlockSpec((1, BN, D), lambda c, i: (c, i, 0)),
        out_shape=jax.ShapeDtypeStruct((B, Nn, D), jnp.float32),
    )(S1, g1, degp3, W2, b1r)


def _tc_readout(S2, g2, degp3, b2r, Bw0T, Bbr):
    B, Nn, D = S2.shape

    def body(s_ref, g_ref, dp_ref, b_ref, bwt_ref, bb_ref, pos_ref, neg_ref):
        dp = dp_ref[...]
        dinv = lax.rsqrt(dp[0] + dp[1] + 1.0)
        bias = b_ref[...]
        Ha = (s_ref[0] + g_ref[0]) * dinv + bias
        Hb = (s_ref[1] + g_ref[1]) * dinv + bias
        m = jnp.mean(Ha, axis=0, keepdims=True)
        srow = jax.nn.sigmoid(m)
        vrow = jnp.dot(srow, bwt_ref[...], preferred_element_type=jnp.float32)
        bb = bb_ref[...]
        vb = vrow.astype(jnp.bfloat16).astype(jnp.float32)
        Hab = Ha.astype(jnp.bfloat16).astype(jnp.float32)
        Hbb = Hb.astype(jnp.bfloat16).astype(jnp.float32)
        pos_ref[...] = jnp.sum(Hab * vb, axis=1, keepdims=True) + bb
        neg_ref[...] = jnp.sum(Hbb * vb, axis=1, keepdims=True) + bb

    return pl.pallas_call(
        body,
        out_shape=(jax.ShapeDtypeStruct((Nn, 1), jnp.float32),
                   jax.ShapeDtypeStruct((Nn, 1), jnp.float32)),
    )(S2, g2, degp3, b2r, Bw0T, Bbr)


@functools.lru_cache(maxsize=None)
def _get_perm(Nn):
    return jax.random.permutation(jax.random.key(42), Nn)


def kernel(x, edge_index, W1, b1, W2, b2, Bw, Bb):
    Nn, D = x.shape
    E = edge_index.shape[1]

    perm = _get_perm(Nn)
    X2 = jnp.stack([x, x[perm]])

    src = edge_index[0]
    dst = edge_index[1]
    nch = E // 16 // _CH
    srcr = src.reshape(16, nch, _CH)
    SRC2 = jnp.stack([srcr, srcr + Nn])
    DSTP = dst.reshape(16, nch, _CH)
    DSTD = dst.reshape(32, E // 32 // _CH, _CH)

    degp = _make_deg(Nn, E)(DSTD)
    degp3 = degp[:, :, None]

    edge_sum = _make_edge_sum(Nn, D, nch)

    g1 = _tc_pre(X2, degp3, W1)
    S1 = edge_sum(g1.reshape(2 * Nn, D), SRC2, DSTP)
    g2 = _tc_mid(S1, g1, degp3, W2, b1.reshape(1, D))
    S2 = edge_sum(g2.reshape(2 * Nn, D), SRC2, DSTP)
    pos, neg = _tc_readout(S2, g2, degp3, b2.reshape(1, D), Bw[0].T,
                           Bb.reshape(1, 1))
    return pos, neg


# speedup vs baseline: 1.0564x; 1.0564x over previous
import functools

import jax
import jax.numpy as jnp
from jax import lax
from jax.experimental import pallas as pl
from jax.experimental.pallas import tpu as pltpu
from jax.experimental.pallas import tpu_sc as plsc

_CH = 125


@functools.lru_cache(maxsize=None)
def _make_deg(Nn, E):
    per_tile = E // 32
    nch = per_tile // _CH
    mesh = plsc.VectorSubcoreMesh(core_axis_name="c", subcore_axis_name="s")

    @functools.partial(
        pl.kernel,
        out_type=jax.ShapeDtypeStruct((2, Nn), jnp.float32),
        mesh=mesh,
        scratch_types=[
            pltpu.VMEM((nch, _CH), jnp.int32),
            pltpu.VMEM((128,), jnp.float32),
            pltpu.VMEM((Nn,), jnp.float32),
            pltpu.VMEM_SHARED((Nn,), jnp.float32),
        ],
    )
    def degk(dst_hbm, out_hbm, dst_v, ones_v, dvmem, dacc):
        c = lax.axis_index("c")
        s = lax.axis_index("s")
        b = c * 16 + s
        pltpu.sync_copy(dst_hbm.at[b], dst_v)
        ov = jnp.ones((16,), jnp.float32)
        for k in range(8):
            ones_v[pl.ds(k * 16, 16)] = ov

        @pl.when(s == 0)
        def _init():
            zv = jnp.zeros((16,), jnp.float32)

            def z(i, carry):
                dvmem[pl.ds(i * 16, 16)] = zv
                return carry

            lax.fori_loop(0, Nn // 16, z, 0)
            pltpu.sync_copy(dvmem, dacc)

        plsc.subcore_barrier()

        def step(j, carry):
            pltpu.sync_copy(ones_v.at[pl.ds(0, _CH)], dacc.at[dst_v.at[j]], add=True)
            return carry

        lax.fori_loop(0, nch, step, 0)
        plsc.subcore_barrier()

        @pl.when(s == 0)
        def _export():
            pltpu.sync_copy(dacc, dvmem)
            pltpu.sync_copy(dvmem, out_hbm.at[c])

    return degk


@functools.lru_cache(maxsize=None)
def _make_edge_sum(Nn, D, nch):
    BLK = 16
    nblk = nch // BLK
    rows_per_tile = Nn // 16
    nz = rows_per_tile // _CH
    mesh = plsc.VectorSubcoreMesh(core_axis_name="c", subcore_axis_name="s")

    @functools.partial(
        pl.kernel,
        out_type=jax.ShapeDtypeStruct((2, Nn, D), jnp.float32),
        mesh=mesh,
        scratch_types=[
            pltpu.VMEM((BLK, _CH), jnp.int32),
            pltpu.VMEM((BLK, _CH), jnp.int32),
            pltpu.VMEM((_CH, D), jnp.float32),
            pltpu.VMEM((_CH, D), jnp.float32),
            pltpu.VMEM_SHARED((Nn, D), jnp.float32),
            pltpu.SemaphoreType.DMA,
            pltpu.SemaphoreType.DMA,
        ],
    )
    def edge_sum(g_hbm, src_hbm, dst_hbm, out_hbm, src_v, dst_v, rows0, rows1,
                 acc, sem0, sem1):
        c = lax.axis_index("c")
        s = lax.axis_index("s")

        zv = jnp.zeros((16,), jnp.float32)

        def zrow(i, carry):
            for k in range(D // 16):
                rows0[i, pl.ds(k * 16, 16)] = zv
            return carry

        lax.fori_loop(0, _CH, zrow, 0)

        def zcp(t, carry):
            pltpu.sync_copy(rows0, acc.at[pl.ds(s * rows_per_tile + t * _CH, _CH)])
            return carry

        lax.fori_loop(0, nz, zcp, 0)
        plsc.subcore_barrier()

        def blk_body(bi, carry):
            pltpu.sync_copy(src_hbm.at[c, s, pl.ds(bi * BLK, BLK)], src_v)
            pltpu.sync_copy(dst_hbm.at[s, pl.ds(bi * BLK, BLK)], dst_v)
            pltpu.async_copy(g_hbm.at[src_v.at[0]], rows0, sem0)
            pltpu.async_copy(g_hbm.at[src_v.at[1]], rows1, sem1)

            def step(i, carry2):
                j = 2 * i
                pltpu.make_async_copy(g_hbm.at[src_v.at[j]], rows0, sem0).wait()
                pltpu.sync_copy(rows0, acc.at[dst_v.at[j]], add=True)

                @pl.when(j + 2 < BLK)
                def _():
                    pltpu.async_copy(g_hbm.at[src_v.at[j + 2]], rows0, sem0)

                pltpu.make_async_copy(g_hbm.at[src_v.at[j + 1]], rows1, sem1).wait()
                pltpu.sync_copy(rows1, acc.at[dst_v.at[j + 1]], add=True)

                @pl.when(j + 3 < BLK)
                def _():
                    pltpu.async_copy(g_hbm.at[src_v.at[j + 3]], rows1, sem1)

                return carry2

            lax.fori_loop(0, BLK // 2, step, 0)
            return carry

        lax.fori_loop(0, nblk, blk_body, 0)
        plsc.subcore_barrier()

        base = s * 624

        def exp_chunk(off, sz):
            pltpu.sync_copy(acc.at[pl.ds(base + off, sz)],
                            rows0.at[pl.ds(0, sz)])
            pltpu.sync_copy(rows0.at[pl.ds(0, sz)],
                            out_hbm.at[c, pl.ds(base + off, sz)])

        for off in (0, 120, 240, 360, 480):
            exp_chunk(off, 120)

        @pl.when(s < 15)
        def _tail24():
            exp_chunk(600, 24)

        @pl.when(s == 15)
        def _tail40():
            exp_chunk(600, 40)

    return edge_sum


def _tc_pre(X2, degp3, W1):
    B, Nn, D = X2.shape
    BN = 1000
    NB = Nn // BN

    def body(x_ref, dp_ref, w_ref, o_ref):
        dp = dp_ref[...]
        dinv = lax.rsqrt(dp[0] + dp[1] + 1.0)
        h = jnp.dot(x_ref[0], w_ref[...], preferred_element_type=jnp.float32)
        o_ref[0] = h * dinv

    return pl.pallas_call(
        body,
        grid=(B, NB),
        in_specs=[
            pl.BlockSpec((1, BN, D), lambda c, i: (c, i, 0)),
            pl.BlockSpec((2, BN, 1), lambda c, i: (0, i, 0)),
            pl.BlockSpec((D, D), lambda c, i: (0, 0)),
        ],
        out_specs=pl.BlockSpec((1, BN, D), lambda c, i: (c, i, 0)),
        out_shape=jax.ShapeDtypeStruct((B, Nn, D), jnp.float32),
    )(X2, degp3, W1)


def _tc_mid(S1, g1, degp3, W2, b1r):
    B, Nn, D = S1.shape
    BN = 1000
    NB = Nn // BN

    def body(s_ref, g_ref, dp_ref, w_ref, b_ref, o_ref):
        dp = dp_ref[...]
        dinv = lax.rsqrt(dp[0] + dp[1] + 1.0)
        t = jnp.maximum((s_ref[0] + g_ref[0]) * dinv + b_ref[...], 0.0)
        h = jnp.dot(t, w_ref[...], preferred_element_type=jnp.float32)
        o_ref[0] = h * dinv

    return pl.pallas_call(
        body,
        grid=(B, NB),
        in_specs=[
            pl.BlockSpec((1, BN, D), lambda c, i: (c, i, 0)),
            pl.BlockSpec((1, BN, D), lambda c, i: (c, i, 0)),
            pl.BlockSpec((2, BN, 1), lambda c, i: (0, i, 0)),
            pl.BlockSpec((D, D), lambda c, i: (0, 0)),
            pl.BlockSpec((1, D), lambda c, i: (0, 0)),
        ],
        out_specs=pl.BlockSpec((1, BN, D), lambda c, i: (c, i, 0)),
        out_shape=jax.ShapeDtypeStruct((B, Nn, D), jnp.float32),
    )(S1, g1, degp3, W2, b1r)


def _tc_readout(S2, g2, degp3, b2r, Bw0T, Bbr):
    B, Nn, D = S2.shape

    def body(s_ref, g_ref, dp_ref, b_ref, bwt_ref, bb_ref, pos_ref, neg_ref):
        dp = dp_ref[...]
        dinv = lax.rsqrt(dp[0] + dp[1] + 1.0)
        bias = b_ref[...]
        Ha = (s_ref[0] + g_ref[0]) * dinv + bias
        Hb = (s_ref[1] + g_ref[1]) * dinv + bias
        m = jnp.mean(Ha, axis=0, keepdims=True)
        srow = jax.nn.sigmoid(m)
        vrow = jnp.dot(srow, bwt_ref[...], preferred_element_type=jnp.float32)
        bb = bb_ref[...]
        vb = vrow.astype(jnp.bfloat16).astype(jnp.float32)
        Hab = Ha.astype(jnp.bfloat16).astype(jnp.float32)
        Hbb = Hb.astype(jnp.bfloat16).astype(jnp.float32)
        pos_ref[...] = jnp.sum(Hab * vb, axis=1, keepdims=True) + bb
        neg_ref[...] = jnp.sum(Hbb * vb, axis=1, keepdims=True) + bb

    return pl.pallas_call(
        body,
        out_shape=(jax.ShapeDtypeStruct((Nn, 1), jnp.float32),
                   jax.ShapeDtypeStruct((Nn, 1), jnp.float32)),
    )(S2, g2, degp3, b2r, Bw0T, Bbr)


@functools.lru_cache(maxsize=None)
def _get_perm(Nn):
    with jax.ensure_compile_time_eval():
        return jax.random.permutation(jax.random.key(42), Nn)


def kernel(x, edge_index, W1, b1, W2, b2, Bw, Bb):
    Nn, D = x.shape
    E = edge_index.shape[1]

    perm = _get_perm(Nn)
    X2 = jnp.stack([x, x[perm]])

    src = edge_index[0]
    dst = edge_index[1]
    nch = E // 16 // _CH
    srcr = src.reshape(16, nch, _CH)
    SRC2 = jnp.stack([srcr, srcr + Nn])
    DSTP = dst.reshape(16, nch, _CH)
    DSTD = dst.reshape(32, E // 32 // _CH, _CH)

    degp = _make_deg(Nn, E)(DSTD)
    degp3 = degp[:, :, None]

    edge_sum = _make_edge_sum(Nn, D, nch)

    g1 = _tc_pre(X2, degp3, W1)
    S1 = edge_sum(g1.reshape(2 * Nn, D), SRC2, DSTP)
    g2 = _tc_mid(S1, g1, degp3, W2, b1.reshape(1, D))
    S2 = edge_sum(g2.reshape(2 * Nn, D), SRC2, DSTP)
    pos, neg = _tc_readout(S2, g2, degp3, b2.reshape(1, D), Bw[0].T,
                           Bb.reshape(1, 1))
    return pos, neg


# speedup vs baseline: 1.0982x; 1.0396x over previous
import functools

import jax
import jax.numpy as jnp
from jax import lax
from jax.experimental import pallas as pl
from jax.experimental.pallas import tpu as pltpu
from jax.experimental.pallas import tpu_sc as plsc

_CH = 125


@functools.lru_cache(maxsize=None)
def _make_deg(Nn, E):
    per_tile = E // 32
    nch = per_tile // _CH
    mesh = plsc.VectorSubcoreMesh(core_axis_name="c", subcore_axis_name="s")

    @functools.partial(
        pl.kernel,
        out_type=jax.ShapeDtypeStruct((2, Nn), jnp.float32),
        mesh=mesh,
        scratch_types=[
            pltpu.VMEM((nch, _CH), jnp.int32),
            pltpu.VMEM((128,), jnp.float32),
            pltpu.VMEM((Nn,), jnp.float32),
            pltpu.VMEM_SHARED((Nn,), jnp.float32),
        ],
    )
    def degk(dst_hbm, out_hbm, dst_v, ones_v, dvmem, dacc):
        c = lax.axis_index("c")
        s = lax.axis_index("s")
        b = c * 16 + s
        pltpu.sync_copy(dst_hbm.at[b], dst_v)
        ov = jnp.ones((16,), jnp.float32)
        for k in range(8):
            ones_v[pl.ds(k * 16, 16)] = ov

        @pl.when(s == 0)
        def _init():
            zv = jnp.zeros((16,), jnp.float32)

            def z(i, carry):
                dvmem[pl.ds(i * 16, 16)] = zv
                return carry

            lax.fori_loop(0, Nn // 16, z, 0)
            pltpu.sync_copy(dvmem, dacc)

        plsc.subcore_barrier()

        def step(j, carry):
            pltpu.sync_copy(ones_v.at[pl.ds(0, _CH)], dacc.at[dst_v.at[j]], add=True)
            return carry

        lax.fori_loop(0, nch, step, 0)
        plsc.subcore_barrier()

        @pl.when(s == 0)
        def _export():
            pltpu.sync_copy(dacc, dvmem)
            pltpu.sync_copy(dvmem, out_hbm.at[c])

    return degk


@functools.lru_cache(maxsize=None)
def _make_edge_sum(Nn, D, nch):
    BLK = 16
    nblk = nch // BLK
    rows_per_tile = Nn // 16
    nz = rows_per_tile // _CH
    mesh = plsc.VectorSubcoreMesh(core_axis_name="c", subcore_axis_name="s")

    @functools.partial(
        pl.kernel,
        out_type=jax.ShapeDtypeStruct((2, Nn, D), jnp.float32),
        mesh=mesh,
        scratch_types=[
            pltpu.VMEM((BLK, _CH), jnp.int32),
            pltpu.VMEM((BLK, _CH), jnp.int32),
            pltpu.VMEM((BLK, _CH), jnp.int32),
            pltpu.VMEM((BLK, _CH), jnp.int32),
            pltpu.VMEM((_CH, D), jnp.float32),
            pltpu.VMEM((_CH, D), jnp.float32),
            pltpu.VMEM_SHARED((Nn, D), jnp.float32),
            pltpu.SemaphoreType.DMA,
            pltpu.SemaphoreType.DMA,
            pltpu.SemaphoreType.DMA,
        ],
    )
    def edge_sum(g_hbm, src_hbm, dst_hbm, out_hbm, src_v0, dst_v0, src_v1,
                 dst_v1, rows0, rows1, acc, sem0, sem1, semi):
        c = lax.axis_index("c")
        s = lax.axis_index("s")

        zv = jnp.zeros((16,), jnp.float32)

        def zrow(i, carry):
            for k in range(D // 16):
                rows0[i, pl.ds(k * 16, 16)] = zv
            return carry

        lax.fori_loop(0, _CH, zrow, 0)

        def zcp(t, carry):
            pltpu.sync_copy(rows0, acc.at[pl.ds(s * rows_per_tile + t * _CH, _CH)])
            return carry

        lax.fori_loop(0, nz, zcp, 0)
        plsc.subcore_barrier()

        def load_blk(bi, sv, dv):
            pltpu.async_copy(src_hbm.at[c, s, pl.ds(bi * BLK, BLK)], sv, semi)
            pltpu.async_copy(dst_hbm.at[s, pl.ds(bi * BLK, BLK)], dv, semi)

        def wait_blk(bi, sv, dv):
            pltpu.make_async_copy(
                src_hbm.at[c, s, pl.ds(bi * BLK, BLK)], sv, semi).wait()
            pltpu.make_async_copy(
                dst_hbm.at[s, pl.ds(bi * BLK, BLK)], dv, semi).wait()

        def proc_blk(sv, dv):
            pltpu.async_copy(g_hbm.at[sv.at[0]], rows0, sem0)
            pltpu.async_copy(g_hbm.at[sv.at[1]], rows1, sem1)

            def step(i, carry2):
                j = 2 * i
                pltpu.make_async_copy(g_hbm.at[sv.at[j]], rows0, sem0).wait()
                pltpu.sync_copy(rows0, acc.at[dv.at[j]], add=True)

                @pl.when(j + 2 < BLK)
                def _():
                    pltpu.async_copy(g_hbm.at[sv.at[j + 2]], rows0, sem0)

                pltpu.make_async_copy(g_hbm.at[sv.at[j + 1]], rows1, sem1).wait()
                pltpu.sync_copy(rows1, acc.at[dv.at[j + 1]], add=True)

                @pl.when(j + 3 < BLK)
                def _():
                    pltpu.async_copy(g_hbm.at[sv.at[j + 3]], rows1, sem1)

                return carry2

            lax.fori_loop(0, BLK // 2, step, 0)

        load_blk(0, src_v0, dst_v0)

        def blk2(k, carry):
            b0 = 2 * k
            wait_blk(b0, src_v0, dst_v0)
            load_blk(b0 + 1, src_v1, dst_v1)
            proc_blk(src_v0, dst_v0)
            wait_blk(b0 + 1, src_v1, dst_v1)

            @pl.when(b0 + 2 < nblk)
            def _():
                load_blk(b0 + 2, src_v0, dst_v0)

            proc_blk(src_v1, dst_v1)
            return carry

        lax.fori_loop(0, nblk // 2, blk2, 0)
        plsc.subcore_barrier()

        base = s * 624

        def exp_chunk(off, sz):
            pltpu.sync_copy(acc.at[pl.ds(base + off, sz)],
                            rows0.at[pl.ds(0, sz)])
            pltpu.sync_copy(rows0.at[pl.ds(0, sz)],
                            out_hbm.at[c, pl.ds(base + off, sz)])

        for off in (0, 120, 240, 360, 480):
            exp_chunk(off, 120)

        @pl.when(s < 15)
        def _tail24():
            exp_chunk(600, 24)

        @pl.when(s == 15)
        def _tail40():
            exp_chunk(600, 40)

    return edge_sum


def _tc_pre(X2, degp3, W1):
    B, Nn, D = X2.shape
    BN = 1000
    NB = Nn // BN

    def body(x_ref, dp_ref, w_ref, o_ref):
        dp = dp_ref[...]
        dinv = lax.rsqrt(dp[0] + dp[1] + 1.0)
        h = jnp.dot(x_ref[0], w_ref[...], preferred_element_type=jnp.float32)
        o_ref[0] = h * dinv

    return pl.pallas_call(
        body,
        grid=(B, NB),
        in_specs=[
            pl.BlockSpec((1, BN, D), lambda c, i: (c, i, 0)),
            pl.BlockSpec((2, BN, 1), lambda c, i: (0, i, 0)),
            pl.BlockSpec((D, D), lambda c, i: (0, 0)),
        ],
        out_specs=pl.BlockSpec((1, BN, D), lambda c, i: (c, i, 0)),
        out_shape=jax.ShapeDtypeStruct((B, Nn, D), jnp.float32),
    )(X2, degp3, W1)


def _tc_mid(S1, g1, degp3, W2, b1r):
    B, Nn, D = S1.shape
    BN = 1000
    NB = Nn // BN

    def body(s_ref, g_ref, dp_ref, w_ref, b_ref, o_ref):
        dp = dp_ref[...]
        dinv = lax.rsqrt(dp[0] + dp[1] + 1.0)
        t = jnp.maximum((s_ref[0] + g_ref[0]) * dinv + b_ref[...], 0.0)
        h = jnp.dot(t, w_ref[...], preferred_element_type=jnp.float32)
        o_ref[0] = h * dinv

    return pl.pallas_call(
        body,
        grid=(B, NB),
        in_specs=[
            pl.BlockSpec((1, BN, D), lambda c, i: (c, i, 0)),
            pl.BlockSpec((1, BN, D), lambda c, i: (c, i, 0)),
            pl.BlockSpec((2, BN, 1), lambda c, i: (0, i, 0)),
            pl.BlockSpec((D, D), lambda c, i: (0, 0)),
            pl.BlockSpec((1, D), lambda c, i: (0, 0)),
        ],
        out_specs=pl.BlockSpec((1, BN, D), lambda c, i: (c, i, 0)),
        out_shape=jax.ShapeDtypeStruct((B, Nn, D), jnp.float32),
    )(S1, g1, degp3, W2, b1r)


def _tc_readout(S2, g2, degp3, b2r, Bw0T, Bbr):
    B, Nn, D = S2.shape

    def body(s_ref, g_ref, dp_ref, b_ref, bwt_ref, bb_ref, pos_ref, neg_ref):
        dp = dp_ref[...]
        dinv = lax.rsqrt(dp[0] + dp[1] + 1.0)
        bias = b_ref[...]
        Ha = (s_ref[0] + g_ref[0]) * dinv + bias
        Hb = (s_ref[1] + g_ref[1]) * dinv + bias
        m = jnp.mean(Ha, axis=0, keepdims=True)
        srow = jax.nn.sigmoid(m)
        vrow = jnp.dot(srow, bwt_ref[...], preferred_element_type=jnp.float32)
        bb = bb_ref[...]
        vb = vrow.astype(jnp.bfloat16).astype(jnp.float32)
        Hab = Ha.astype(jnp.bfloat16).astype(jnp.float32)
        Hbb = Hb.astype(jnp.bfloat16).astype(jnp.float32)
        pos_ref[...] = jnp.sum(Hab * vb, axis=1, keepdims=True) + bb
        neg_ref[...] = jnp.sum(Hbb * vb, axis=1, keepdims=True) + bb

    return pl.pallas_call(
        body,
        out_shape=(jax.ShapeDtypeStruct((Nn, 1), jnp.float32),
                   jax.ShapeDtypeStruct((Nn, 1), jnp.float32)),
    )(S2, g2, degp3, b2r, Bw0T, Bbr)


@functools.lru_cache(maxsize=None)
def _get_perm(Nn):
    with jax.ensure_compile_time_eval():
        return jax.random.permutation(jax.random.key(42), Nn)


def kernel(x, edge_index, W1, b1, W2, b2, Bw, Bb):
    Nn, D = x.shape
    E = edge_index.shape[1]

    perm = _get_perm(Nn)
    X2 = jnp.stack([x, x[perm]])

    src = edge_index[0]
    dst = edge_index[1]
    nch = E // 16 // _CH
    srcr = src.reshape(16, nch, _CH)
    SRC2 = jnp.stack([srcr, srcr + Nn])
    DSTP = dst.reshape(16, nch, _CH)
    DSTD = dst.reshape(32, E // 32 // _CH, _CH)

    degp = _make_deg(Nn, E)(DSTD)
    degp3 = degp[:, :, None]

    edge_sum = _make_edge_sum(Nn, D, nch)

    g1 = _tc_pre(X2, degp3, W1)
    S1 = edge_sum(g1.reshape(2 * Nn, D), SRC2, DSTP)
    g2 = _tc_mid(S1, g1, degp3, W2, b1.reshape(1, D))
    S2 = edge_sum(g2.reshape(2 * Nn, D), SRC2, DSTP)
    pos, neg = _tc_readout(S2, g2, degp3, b2.reshape(1, D), Bw[0].T,
                           Bb.reshape(1, 1))
    return pos, neg
